# baseline (speedup 1.0000x reference)
import functools

import numpy as np

import jax
import jax.numpy as jnp
from jax import lax
from jax.experimental import pallas as pl
from jax.experimental.pallas import tpu as pltpu
from jax.experimental.pallas import tpu_sc as plsc

B, HW, D, NPQ = 16, 1024, 96, 256
NQ, NP = 8, 16
NBQ = B * NQ
NROWS = B * NQ * NP
NW = 32
ROWS_PER_W = NROWS // NW

_RAND11 = np.array([
    [196, 18, 183, 193, 653, 363, 385, 295],
    [6, 258, 552, 1010, 409, 475, 972, 786],
    [587, 898, 835, 519, 566, 651, 268, 707],
    [108, 529, 1008, 539, 284, 311, 261, 676],
    [469, 46, 51, 20, 814, 946, 849, 1005],
    [775, 580, 663, 381, 889, 192, 316, 676],
    [803, 525, 660, 731, 978, 371, 1016, 439],
    [11, 338, 859, 953, 793, 774, 800, 648],
    [643, 377, 308, 608, 578, 185, 172, 837],
    [1011, 45, 676, 508, 302, 938, 561, 97],
    [535, 720, 437, 812, 433, 824, 856, 56],
    [424, 1022, 95, 661, 830, 696, 147, 985],
    [1015, 479, 186, 993, 817, 348, 293, 548],
    [127, 460, 574, 546, 665, 153, 891, 1023],
    [291, 700, 321, 611, 389, 264, 862, 611],
    [643, 832, 258, 67, 354, 212, 206, 902],
], dtype=np.int32)
_GIDX = [int(v) for v in (_RAND11 + np.arange(B)[:, None] * HW).reshape(-1)]


def _attn_topk_body(z_ref, zp_ref, idx_ref):
    dots = []
    for b in range(B):
        rows = jnp.concatenate(
            [z_ref[b, int(r)][None, :] for r in _RAND11[b]], axis=0)
        dots.append(lax.dot_general(rows, zp_ref[b], (((1,), (1,)), ((), ())),
                                    preferred_element_type=jnp.float32))
    attn = jnp.concatenate(dots, axis=0)
    attn = attn.T
    iota = lax.broadcasted_iota(jnp.int32, (HW, NBQ), 0)
    sels = []
    pv = pi = None
    for k in range(NP):
        if k == 0:
            m = jnp.max(attn, axis=0, keepdims=True)
            cand = jnp.where(attn == m, iota, HW)
        else:
            ok = (attn < pv) | ((attn == pv) & (iota > pi))
            m = jnp.max(jnp.where(ok, attn, -jnp.inf), axis=0, keepdims=True)
            cand = jnp.where((attn == m) & ok, iota, HW)
        sel = jnp.min(cand, axis=0, keepdims=True)
        sels.append(sel)
        pv, pi = m, sel
    r = jnp.concatenate(sels, axis=0)
    base = lax.broadcasted_iota(jnp.int32, (NP, NBQ), 1) // NQ * HW
    idx_ref[...] = r + base


def _attn_topk(z3, zp):
    return pl.pallas_call(
        _attn_topk_body,
        out_shape=jax.ShapeDtypeStruct((NP, NBQ), jnp.int32),
    )(z3, zp)


_SROWS_PER_W = NQ * B // (NW // 2)


def _sc_gather(table, idx, table2, idx2):
    @functools.partial(
        pl.kernel,
        mesh=plsc.VectorSubcoreMesh(core_axis_name="c", subcore_axis_name="s"),
        out_type=(jax.ShapeDtypeStruct((NROWS, NPQ), jnp.float32),
                  jax.ShapeDtypeStruct((NQ * B, NPQ), jnp.float32)),
        scratch_types=[
            pltpu.VMEM((ROWS_PER_W,), jnp.int32),
            pltpu.VMEM((ROWS_PER_W, NPQ), jnp.float32),
            pltpu.VMEM((_SROWS_PER_W,), jnp.int32),
            pltpu.VMEM((_SROWS_PER_W, NPQ), jnp.float32),
            pltpu.SemaphoreType.DMA,
        ],
    )
    def gather_kernel(table_hbm, idx_hbm, table2_hbm, idx2_hbm,
                      out_hbm, out2_hbm, idx_v, rows_v, idx2_v, rows2_v, sem):
        s = lax.axis_index("s")
        c = lax.axis_index("c")
        w = s * 2 + c

        @pl.when(w < NW // 2)
        def _sample():
            pltpu.sync_copy(idx2_hbm.at[pl.ds(w * _SROWS_PER_W,
                                              _SROWS_PER_W)], idx2_v)
            pltpu.async_copy(table2_hbm.at[idx2_v], rows2_v, sem).wait()
            pltpu.sync_copy(rows2_v,
                            out2_hbm.at[pl.ds(w * _SROWS_PER_W,
                                              _SROWS_PER_W)])

        pltpu.sync_copy(idx_hbm.at[s, pl.ds(c * ROWS_PER_W, ROWS_PER_W)],
                        idx_v)
        pltpu.async_copy(table_hbm.at[idx_v], rows_v, sem).wait()
        pltpu.sync_copy(rows_v, out_hbm.at[pl.ds(w * ROWS_PER_W, ROWS_PER_W)])

    return gather_kernel(table, idx, table2, idx2)


def _jsd_body(p_ref, g_ref, out_ref):
    acc = jnp.zeros((), jnp.float32)
    for pos in range(NP):
        p16 = p_ref[pos % NQ]
        q = g_ref[pos]
        p = jnp.broadcast_to(p16[:, None, :], (B, NQ, NPQ)).reshape(NBQ, NPQ)
        m = jnp.log(jnp.clip((p + q) * 0.5, 1e-7, 1.0))
        tp = jnp.where(p > 0, p * (jnp.log(jnp.where(p > 0, p, 1.0)) - m), 0.0)
        tq = jnp.where(q > 0, q * (jnp.log(jnp.where(q > 0, q, 1.0)) - m), 0.0)
        acc = acc + jnp.sum(tp + tq)
    out_ref[...] = (acc * jnp.float32(0.5 / NROWS))[None, None]


def _jsd(sample_z_dis_t, gathered):
    return pl.pallas_call(
        _jsd_body,
        out_shape=jax.ShapeDtypeStruct((1, 1), jnp.float32),
    )(sample_z_dis_t, gathered)


_GIDX_ARR = np.array(_GIDX, dtype=np.int32)
_GIDX_T_ARR = _GIDX_ARR.reshape(B, NQ).T.reshape(-1).copy()


def kernel(z, z_pos, z_dis, z_pos_dis):
    idx = _attn_topk(z.reshape(B, HW, D),
                     z_pos.reshape(B, HW, D))
    gathered, sample_z_dis_t = _sc_gather(
        z_pos_dis.reshape(B * HW, NPQ), idx,
        z_dis.reshape(B * HW, NPQ), jnp.asarray(_GIDX_T_ARR))
    partial = _jsd(sample_z_dis_t.reshape(NQ, B, NPQ),
                   gathered.reshape(NP, NBQ, NPQ))
    return partial[0, 0]


# speedup vs baseline: 1.0442x; 1.0442x over previous
import functools

import numpy as np

import jax
import jax.numpy as jnp
from jax import lax
from jax.experimental import pallas as pl
from jax.experimental.pallas import tpu as pltpu
from jax.experimental.pallas import tpu_sc as plsc

B, HW, D, NPQ = 16, 1024, 96, 256
NQ, NP = 8, 16
NBQ = B * NQ
NROWS = B * NQ * NP
NW = 32
ROWS_PER_W = NROWS // NW

_RAND11 = np.array([
    [196, 18, 183, 193, 653, 363, 385, 295],
    [6, 258, 552, 1010, 409, 475, 972, 786],
    [587, 898, 835, 519, 566, 651, 268, 707],
    [108, 529, 1008, 539, 284, 311, 261, 676],
    [469, 46, 51, 20, 814, 946, 849, 1005],
    [775, 580, 663, 381, 889, 192, 316, 676],
    [803, 525, 660, 731, 978, 371, 1016, 439],
    [11, 338, 859, 953, 793, 774, 800, 648],
    [643, 377, 308, 608, 578, 185, 172, 837],
    [1011, 45, 676, 508, 302, 938, 561, 97],
    [535, 720, 437, 812, 433, 824, 856, 56],
    [424, 1022, 95, 661, 830, 696, 147, 985],
    [1015, 479, 186, 993, 817, 348, 293, 548],
    [127, 460, 574, 546, 665, 153, 891, 1023],
    [291, 700, 321, 611, 389, 264, 862, 611],
    [643, 832, 258, 67, 354, 212, 206, 902],
], dtype=np.int32)
_GIDX = [int(v) for v in (_RAND11 + np.arange(B)[:, None] * HW).reshape(-1)]


def _attn_topk_body(z_ref, zp_ref, idx_ref):
    dots = []
    for b in range(B):
        rows = jnp.concatenate(
            [z_ref[b, int(r)][None, :] for r in _RAND11[b]], axis=0)
        dots.append(lax.dot_general(rows, zp_ref[b], (((1,), (1,)), ((), ())),
                                    preferred_element_type=jnp.float32))
    attn = jnp.concatenate(dots, axis=0)
    attn = attn.T
    iota = lax.broadcasted_iota(jnp.int32, (HW, NBQ), 0)
    sels = []
    for _ in range(NP):
        m = jnp.max(attn, axis=0, keepdims=True)
        cand = jnp.where(attn == m, iota, HW)
        sel = jnp.min(cand, axis=0, keepdims=True)
        sels.append(sel)
        attn = jnp.where(iota == sel, -jnp.inf, attn)
    r = jnp.concatenate(sels, axis=0)
    base = lax.broadcasted_iota(jnp.int32, (NP, NBQ), 1) // NQ * HW
    idx_ref[...] = r + base


def _attn_topk(z3, zp):
    return pl.pallas_call(
        _attn_topk_body,
        out_shape=jax.ShapeDtypeStruct((NP, NBQ), jnp.int32),
    )(z3, zp)


_SROWS_PER_W = NQ * B // (NW // 2)


def _sc_gather(table, idx, table2, idx2):
    @functools.partial(
        pl.kernel,
        mesh=plsc.VectorSubcoreMesh(core_axis_name="c", subcore_axis_name="s"),
        out_type=(jax.ShapeDtypeStruct((NROWS, NPQ), jnp.float32),
                  jax.ShapeDtypeStruct((NQ * B, NPQ), jnp.float32)),
        scratch_types=[
            pltpu.VMEM((ROWS_PER_W,), jnp.int32),
            pltpu.VMEM((ROWS_PER_W, NPQ), jnp.float32),
            pltpu.VMEM((_SROWS_PER_W,), jnp.int32),
            pltpu.VMEM((_SROWS_PER_W, NPQ), jnp.float32),
            pltpu.SemaphoreType.DMA,
            pltpu.SemaphoreType.DMA,
        ],
    )
    def gather_kernel(table_hbm, idx_hbm, table2_hbm, idx2_hbm,
                      out_hbm, out2_hbm, idx_v, rows_v, idx2_v, rows2_v,
                      sem, sem2):
        s = lax.axis_index("s")
        c = lax.axis_index("c")
        w = s * 2 + c

        pltpu.sync_copy(idx_hbm.at[s, pl.ds(c * ROWS_PER_W, ROWS_PER_W)],
                        idx_v)
        main = pltpu.async_copy(table_hbm.at[idx_v], rows_v, sem)

        @pl.when(w < NW // 2)
        def _sample():
            pltpu.sync_copy(idx2_hbm.at[pl.ds(w * _SROWS_PER_W,
                                              _SROWS_PER_W)], idx2_v)
            pltpu.async_copy(table2_hbm.at[idx2_v], rows2_v, sem2).wait()
            pltpu.sync_copy(rows2_v,
                            out2_hbm.at[pl.ds(w * _SROWS_PER_W,
                                              _SROWS_PER_W)])

        main.wait()
        pltpu.sync_copy(rows_v, out_hbm.at[pl.ds(w * ROWS_PER_W, ROWS_PER_W)])

    return gather_kernel(table, idx, table2, idx2)


def _jsd_body(p_ref, g_ref, out_ref):
    acc = jnp.zeros((), jnp.float32)
    for pos in range(NP):
        p16 = p_ref[pos % NQ]
        q = g_ref[pos]
        p = jnp.broadcast_to(p16[:, None, :], (B, NQ, NPQ)).reshape(NBQ, NPQ)
        m = jnp.log(jnp.clip((p + q) * 0.5, 1e-7, 1.0))
        tp = jnp.where(p > 0, p * (jnp.log(jnp.where(p > 0, p, 1.0)) - m), 0.0)
        tq = jnp.where(q > 0, q * (jnp.log(jnp.where(q > 0, q, 1.0)) - m), 0.0)
        acc = acc + jnp.sum(tp + tq)
    out_ref[...] = (acc * jnp.float32(0.5 / NROWS))[None, None]


def _jsd(sample_z_dis_t, gathered):
    return pl.pallas_call(
        _jsd_body,
        out_shape=jax.ShapeDtypeStruct((1, 1), jnp.float32),
    )(sample_z_dis_t, gathered)


_GIDX_ARR = np.array(_GIDX, dtype=np.int32)
_GIDX_T_ARR = _GIDX_ARR.reshape(B, NQ).T.reshape(-1).copy()


def kernel(z, z_pos, z_dis, z_pos_dis):
    idx = _attn_topk(z.reshape(B, HW, D),
                     z_pos.reshape(B, HW, D))
    gathered, sample_z_dis_t = _sc_gather(
        z_pos_dis.reshape(B * HW, NPQ), idx,
        z_dis.reshape(B * HW, NPQ), jnp.asarray(_GIDX_T_ARR))
    partial = _jsd(sample_z_dis_t.reshape(NQ, B, NPQ),
                   gathered.reshape(NP, NBQ, NPQ))
    return partial[0, 0]


# speedup vs baseline: 1.0586x; 1.0137x over previous
import functools

import numpy as np

import jax
import jax.numpy as jnp
from jax import lax
from jax.experimental import pallas as pl
from jax.experimental.pallas import tpu as pltpu
from jax.experimental.pallas import tpu_sc as plsc

B, HW, D, NPQ = 16, 1024, 96, 256
NQ, NP = 8, 16
NBQ = B * NQ
NROWS = B * NQ * NP
NW = 32
ROWS_PER_W = NROWS // NW

_RAND11 = np.array([
    [196, 18, 183, 193, 653, 363, 385, 295],
    [6, 258, 552, 1010, 409, 475, 972, 786],
    [587, 898, 835, 519, 566, 651, 268, 707],
    [108, 529, 1008, 539, 284, 311, 261, 676],
    [469, 46, 51, 20, 814, 946, 849, 1005],
    [775, 580, 663, 381, 889, 192, 316, 676],
    [803, 525, 660, 731, 978, 371, 1016, 439],
    [11, 338, 859, 953, 793, 774, 800, 648],
    [643, 377, 308, 608, 578, 185, 172, 837],
    [1011, 45, 676, 508, 302, 938, 561, 97],
    [535, 720, 437, 812, 433, 824, 856, 56],
    [424, 1022, 95, 661, 830, 696, 147, 985],
    [1015, 479, 186, 993, 817, 348, 293, 548],
    [127, 460, 574, 546, 665, 153, 891, 1023],
    [291, 700, 321, 611, 389, 264, 862, 611],
    [643, 832, 258, 67, 354, 212, 206, 902],
], dtype=np.int32)
_GIDX = [int(v) for v in (_RAND11 + np.arange(B)[:, None] * HW).reshape(-1)]


_GSTEPS = 2
_BPG = B // _GSTEPS


def _attn_topk_body(z_ref, zp_ref, idx_ref, attn_ref):
    g = pl.program_id(0)
    for gg in range(_GSTEPS):
        @pl.when(g == gg)
        def _matmul(gg=gg):
            dots = []
            for bl in range(_BPG):
                b = gg * _BPG + bl
                rows = jnp.concatenate(
                    [z_ref[bl, int(r)][None, :] for r in _RAND11[b]], axis=0)
                dots.append(lax.dot_general(
                    rows, zp_ref[bl], (((1,), (1,)), ((), ())),
                    preferred_element_type=jnp.float32))
            attn_ref[pl.ds(gg * _BPG * NQ, _BPG * NQ), :] = jnp.concatenate(
                dots, axis=0)

    @pl.when(g == _GSTEPS - 1)
    def _topk():
        _topk_phase(attn_ref, idx_ref)


def _topk_phase(attn_ref, idx_ref):
    attn = attn_ref[...].T
    iota = lax.broadcasted_iota(jnp.int32, (HW, NBQ), 0)
    sels = []
    for _ in range(NP):
        m = jnp.max(attn, axis=0, keepdims=True)
        cand = jnp.where(attn == m, iota, HW)
        sel = jnp.min(cand, axis=0, keepdims=True)
        sels.append(sel)
        attn = jnp.where(iota == sel, -jnp.inf, attn)
    r = jnp.concatenate(sels, axis=0)
    base = lax.broadcasted_iota(jnp.int32, (NP, NBQ), 1) // NQ * HW
    idx_ref[...] = r + base


def _attn_topk(z3, zp):
    return pl.pallas_call(
        _attn_topk_body,
        grid=(_GSTEPS,),
        in_specs=[
            pl.BlockSpec((_BPG, HW, D), lambda g: (g, 0, 0)),
            pl.BlockSpec((_BPG, HW, D), lambda g: (g, 0, 0)),
        ],
        out_specs=pl.BlockSpec((NP, NBQ), lambda g: (0, 0)),
        out_shape=jax.ShapeDtypeStruct((NP, NBQ), jnp.int32),
        scratch_shapes=[pltpu.VMEM((NBQ, HW), jnp.float32)],
    )(z3, zp)


_SROWS_PER_W = NQ * B // (NW // 2)


def _sc_gather(table, idx, table2, idx2):
    @functools.partial(
        pl.kernel,
        mesh=plsc.VectorSubcoreMesh(core_axis_name="c", subcore_axis_name="s"),
        out_type=(jax.ShapeDtypeStruct((NROWS, NPQ), jnp.float32),
                  jax.ShapeDtypeStruct((NQ * B, NPQ), jnp.float32)),
        scratch_types=[
            pltpu.VMEM((ROWS_PER_W,), jnp.int32),
            pltpu.VMEM((ROWS_PER_W, NPQ), jnp.float32),
            pltpu.VMEM((_SROWS_PER_W,), jnp.int32),
            pltpu.VMEM((_SROWS_PER_W, NPQ), jnp.float32),
            pltpu.SemaphoreType.DMA,
            pltpu.SemaphoreType.DMA,
        ],
    )
    def gather_kernel(table_hbm, idx_hbm, table2_hbm, idx2_hbm,
                      out_hbm, out2_hbm, idx_v, rows_v, idx2_v, rows2_v,
                      sem, sem2):
        s = lax.axis_index("s")
        c = lax.axis_index("c")
        w = s * 2 + c

        pltpu.sync_copy(idx_hbm.at[s, pl.ds(c * ROWS_PER_W, ROWS_PER_W)],
                        idx_v)
        main = pltpu.async_copy(table_hbm.at[idx_v], rows_v, sem)

        @pl.when(w < NW // 2)
        def _sample():
            pltpu.sync_copy(idx2_hbm.at[pl.ds(w * _SROWS_PER_W,
                                              _SROWS_PER_W)], idx2_v)
            pltpu.async_copy(table2_hbm.at[idx2_v], rows2_v, sem2).wait()
            pltpu.sync_copy(rows2_v,
                            out2_hbm.at[pl.ds(w * _SROWS_PER_W,
                                              _SROWS_PER_W)])

        main.wait()
        pltpu.sync_copy(rows_v, out_hbm.at[pl.ds(w * ROWS_PER_W, ROWS_PER_W)])

    return gather_kernel(table, idx, table2, idx2)


def _jsd_body(p_ref, g_ref, out_ref):
    acc = jnp.zeros((), jnp.float32)
    for pos in range(NP):
        p16 = p_ref[pos % NQ]
        q = g_ref[pos]
        p = jnp.broadcast_to(p16[:, None, :], (B, NQ, NPQ)).reshape(NBQ, NPQ)
        m = jnp.log(jnp.clip((p + q) * 0.5, 1e-7, 1.0))
        tp = jnp.where(p > 0, p * (jnp.log(jnp.where(p > 0, p, 1.0)) - m), 0.0)
        tq = jnp.where(q > 0, q * (jnp.log(jnp.where(q > 0, q, 1.0)) - m), 0.0)
        acc = acc + jnp.sum(tp + tq)
    out_ref[...] = (acc * jnp.float32(0.5 / NROWS))[None, None]


def _jsd(sample_z_dis_t, gathered):
    return pl.pallas_call(
        _jsd_body,
        out_shape=jax.ShapeDtypeStruct((1, 1), jnp.float32),
    )(sample_z_dis_t, gathered)


_GIDX_ARR = np.array(_GIDX, dtype=np.int32)
_GIDX_T_ARR = _GIDX_ARR.reshape(B, NQ).T.reshape(-1).copy()


def kernel(z, z_pos, z_dis, z_pos_dis):
    idx = _attn_topk(z.reshape(B, HW, D),
                     z_pos.reshape(B, HW, D))
    gathered, sample_z_dis_t = _sc_gather(
        z_pos_dis.reshape(B * HW, NPQ), idx,
        z_dis.reshape(B * HW, NPQ), jnp.asarray(_GIDX_T_ARR))
    partial = _jsd(sample_z_dis_t.reshape(NQ, B, NPQ),
                   gathered.reshape(NP, NBQ, NPQ))
    return partial[0, 0]


# speedup vs baseline: 1.0598x; 1.0012x over previous
import functools

import numpy as np

import jax
import jax.numpy as jnp
from jax import lax
from jax.experimental import pallas as pl
from jax.experimental.pallas import tpu as pltpu
from jax.experimental.pallas import tpu_sc as plsc

B, HW, D, NPQ = 16, 1024, 96, 256
NQ, NP = 8, 16
NBQ = B * NQ
NROWS = B * NQ * NP
NW = 32
ROWS_PER_W = NROWS // NW

_RAND11 = np.array([
    [196, 18, 183, 193, 653, 363, 385, 295],
    [6, 258, 552, 1010, 409, 475, 972, 786],
    [587, 898, 835, 519, 566, 651, 268, 707],
    [108, 529, 1008, 539, 284, 311, 261, 676],
    [469, 46, 51, 20, 814, 946, 849, 1005],
    [775, 580, 663, 381, 889, 192, 316, 676],
    [803, 525, 660, 731, 978, 371, 1016, 439],
    [11, 338, 859, 953, 793, 774, 800, 648],
    [643, 377, 308, 608, 578, 185, 172, 837],
    [1011, 45, 676, 508, 302, 938, 561, 97],
    [535, 720, 437, 812, 433, 824, 856, 56],
    [424, 1022, 95, 661, 830, 696, 147, 985],
    [1015, 479, 186, 993, 817, 348, 293, 548],
    [127, 460, 574, 546, 665, 153, 891, 1023],
    [291, 700, 321, 611, 389, 264, 862, 611],
    [643, 832, 258, 67, 354, 212, 206, 902],
], dtype=np.int32)
_GIDX = [int(v) for v in (_RAND11 + np.arange(B)[:, None] * HW).reshape(-1)]


_GSTEPS = 2
_BPG = B // _GSTEPS


def _attn_topk_body(z_ref, zp_ref, idx_ref, attn_ref):
    g = pl.program_id(0)
    for gg in range(_GSTEPS):
        @pl.when(g == gg)
        def _matmul(gg=gg):
            dots = []
            for bl in range(_BPG):
                b = gg * _BPG + bl
                rows = jnp.concatenate(
                    [z_ref[bl, int(r)][None, :] for r in _RAND11[b]], axis=0)
                dots.append(lax.dot_general(
                    rows, zp_ref[bl], (((1,), (1,)), ((), ())),
                    preferred_element_type=jnp.float32))
            attn_ref[pl.ds(gg * _BPG * NQ, _BPG * NQ), :] = jnp.concatenate(
                dots, axis=0)

    @pl.when(g == _GSTEPS - 1)
    def _topk():
        _topk_phase(attn_ref, idx_ref)


def _topk_phase(attn_ref, idx_ref):
    attn = attn_ref[...].T
    iota = lax.broadcasted_iota(jnp.int32, (HW, NBQ), 0)
    sels = []
    for _ in range(NP):
        m = jnp.max(attn, axis=0, keepdims=True)
        cand = jnp.where(attn == m, iota, HW)
        sel = jnp.min(cand, axis=0, keepdims=True)
        sels.append(sel)
        attn = jnp.where(iota == sel, -jnp.inf, attn)
    r = jnp.concatenate(sels, axis=0)
    base = lax.broadcasted_iota(jnp.int32, (NP, NBQ), 1) // NQ * HW
    idx_ref[...] = r + base


def _attn_topk(z3, zp):
    return pl.pallas_call(
        _attn_topk_body,
        grid=(_GSTEPS,),
        in_specs=[
            pl.BlockSpec((_BPG, HW, D), lambda g: (g, 0, 0)),
            pl.BlockSpec((_BPG, HW, D), lambda g: (g, 0, 0)),
        ],
        out_specs=pl.BlockSpec((NP, NBQ), lambda g: (0, 0)),
        out_shape=jax.ShapeDtypeStruct((NP, NBQ), jnp.int32),
        scratch_shapes=[pltpu.VMEM((NBQ, HW), jnp.float32)],
    )(z3, zp)


_SROWS_PER_W = NQ * B // (NW // 2)


def _sc_gather(table, idx, table2, idx2):
    @functools.partial(
        pl.kernel,
        mesh=plsc.VectorSubcoreMesh(core_axis_name="c", subcore_axis_name="s"),
        out_type=(jax.ShapeDtypeStruct((NROWS, NPQ), jnp.float32),
                  jax.ShapeDtypeStruct((NQ * B, NPQ), jnp.float32)),
        scratch_types=[
            pltpu.VMEM((ROWS_PER_W,), jnp.int32),
            pltpu.VMEM((ROWS_PER_W, NPQ), jnp.float32),
            pltpu.VMEM((_SROWS_PER_W,), jnp.int32),
            pltpu.VMEM((_SROWS_PER_W, NPQ), jnp.float32),
            pltpu.SemaphoreType.DMA,
            pltpu.SemaphoreType.DMA,
        ],
    )
    def gather_kernel(table_hbm, idx_hbm, table2_hbm, idx2_hbm,
                      out_hbm, out2_hbm, idx_v, rows_v, idx2_v, rows2_v,
                      sem, sem2):
        s = lax.axis_index("s")
        c = lax.axis_index("c")
        w = s * 2 + c

        pltpu.sync_copy(idx_hbm.at[s, pl.ds(c * ROWS_PER_W, ROWS_PER_W)],
                        idx_v)
        main = pltpu.async_copy(table_hbm.at[idx_v], rows_v, sem)

        @pl.when(w < NW // 2)
        def _sample():
            pltpu.sync_copy(idx2_hbm.at[pl.ds(w * _SROWS_PER_W,
                                              _SROWS_PER_W)], idx2_v)
            pltpu.async_copy(table2_hbm.at[idx2_v], rows2_v, sem2).wait()
            pltpu.sync_copy(rows2_v,
                            out2_hbm.at[pl.ds(w * _SROWS_PER_W,
                                              _SROWS_PER_W)])

        main.wait()
        pltpu.sync_copy(rows_v, out_hbm.at[pl.ds(w * ROWS_PER_W, ROWS_PER_W)])

    return gather_kernel(table, idx, table2, idx2)


def _jsd_body(p_ref, g_ref, out_ref):
    acc = jnp.zeros((), jnp.float32)
    for pos in range(NP):
        p16 = p_ref[pos % NQ]
        q = g_ref[pos]
        p = jnp.broadcast_to(p16[:, None, :], (B, NQ, NPQ)).reshape(NBQ, NPQ)
        m = jnp.log(jnp.clip((p + q) * 0.5, 1e-7, 1.0))
        tp = jnp.where(p > 0, p * (jnp.log(jnp.where(p > 0, p, 1.0)) - m), 0.0)
        tq = jnp.where(q > 0, q * (jnp.log(jnp.where(q > 0, q, 1.0)) - m), 0.0)
        acc = acc + jnp.sum(tp + tq)
    out_ref[...] = (acc * jnp.float32(0.5 / NROWS))[None, None]


def _jsd(sample_z_dis_t, gathered):
    return pl.pallas_call(
        _jsd_body,
        out_shape=jax.ShapeDtypeStruct((1, 1), jnp.float32),
    )(sample_z_dis_t, gathered)


_GIDX_T_ARR = np.array(_GIDX, dtype=np.int32).reshape(B, NQ).T.reshape(-1).copy()


def kernel(z, z_pos, z_dis, z_pos_dis):
    idx = _attn_topk(z.reshape(B, HW, D),
                     z_pos.reshape(B, HW, D))
    gathered, sample_z_dis_t = _sc_gather(
        z_pos_dis.reshape(B * HW, NPQ), idx,
        z_dis.reshape(B * HW, NPQ), jnp.asarray(_GIDX_T_ARR))
    partial = _jsd(sample_z_dis_t.reshape(NQ, B, NPQ),
                   gathered.reshape(NP, NBQ, NPQ))
    return partial[0, 0]
